# Initial kernel scaffold; baseline (speedup 1.0000x reference)
import jax
import jax.numpy as jnp
from jax.experimental import pallas as pl


def kernel(x, edge_index, simi_weight, W_neigh, b_neigh, W_self, b_self):
    raise NotImplementedError("write your pallas kernel here")


# speedup vs baseline: 18.0834x; 18.0834x over previous
import functools

import jax
import jax.numpy as jnp
from jax import lax
from jax.experimental import pallas as pl
from jax.experimental.pallas import tpu as pltpu
from jax.experimental.pallas import tpu_sc as plsc

_B = 80


def _tc_linear_body(x_ref, w_ref, b_ref, o_ref):
    o_ref[...] = (
        jnp.dot(x_ref[...], w_ref[...], preferred_element_type=jnp.float32)
        + b_ref[...]
    )


def _tc_combine_body(x_ref, w_ref, b_ref, p0_ref, p1_ref, o_ref):
    o_ref[...] = (
        jnp.dot(x_ref[...], w_ref[...], preferred_element_type=jnp.float32)
        + b_ref[...]
        + p0_ref[...]
        + p1_ref[...]
    )


def _make_sc_kernel(n, e, d):
    edges_per_sc_tile = e // 16
    edges_per_tile = e // 32
    nb1 = edges_per_sc_tile // _B
    nb3 = edges_per_tile // _B
    rpt = (n // 16) // 8 * 8
    rpt_last = n - 15 * rpt
    zc = 640

    mesh = plsc.VectorSubcoreMesh(core_axis_name="c", subcore_axis_name="s")

    @functools.partial(
        pl.kernel,
        out_type=jax.ShapeDtypeStruct((2 * n, d), jnp.float32),
        mesh=mesh,
        compiler_params=pltpu.CompilerParams(
            needs_layout_passes=False, use_tc_tiling_on_sc=False
        ),
        scratch_types=[
            pltpu.VMEM_SHARED((n, d), jnp.float32),
            pltpu.VMEM_SHARED((n,), jnp.float32),
            pltpu.VMEM_SHARED((n,), jnp.float32),
            pltpu.VMEM((n,), jnp.float32),
            pltpu.VMEM((n,), jnp.float32),
            pltpu.VMEM((n,), jnp.float32),
            pltpu.VMEM((_B,), jnp.int32),
            pltpu.VMEM((_B,), jnp.int32),
            pltpu.VMEM((_B,), jnp.float32),
            pltpu.VMEM((_B,), jnp.float32),
            pltpu.VMEM((_B, d), jnp.float32),
            pltpu.SemaphoreType.DMA,
        ],
    )
    def sc_kernel(h_hbm, src_hbm, dst_hbm, simi_hbm, out_hbm,
                  acc, bins_h, bins_t, simi_v, bh_v, bt_v,
                  sidx, didx, ebh, ebt, rows, sem):
        c = lax.axis_index("c")
        s = lax.axis_index("s")
        zeros16 = jnp.zeros((16,), jnp.float32)

        def zero_rows(i, _):
            for j in range(d // 16):
                rows[i, pl.ds(j * 16, 16)] = zeros16
            return _

        lax.fori_loop(0, _B, zero_rows, None)

        def zero_bh(i, _):
            bh_v[pl.ds(i * 16, 16)] = zeros16
            return _

        lax.fori_loop(0, zc // 16, zero_bh, None)

        row0 = s * rpt

        @pl.when(s < 15)
        def _():
            for k in range(rpt // _B):
                pltpu.sync_copy(rows, acc.at[pl.ds(row0 + k * _B, _B), :])
            tail = rpt - (rpt // _B) * _B
            if tail:
                pltpu.sync_copy(rows.at[pl.ds(0, tail), :],
                                acc.at[pl.ds(row0 + (rpt // _B) * _B, tail), :])

        @pl.when(s == 15)
        def _():
            for k in range(rpt_last // _B):
                pltpu.sync_copy(rows, acc.at[pl.ds(15 * rpt + k * _B, _B), :])

        @pl.when(s < 15)
        def _():
            pltpu.sync_copy(bh_v.at[pl.ds(0, zc)], bins_h.at[pl.ds(s * zc, zc)])
            pltpu.sync_copy(bh_v.at[pl.ds(0, zc)], bins_t.at[pl.ds(s * zc, zc)])

        @pl.when(s == 15)
        def _():
            rem = n - 15 * zc
            pltpu.sync_copy(bh_v.at[pl.ds(0, rem)], bins_h.at[pl.ds(15 * zc, rem)])
            pltpu.sync_copy(bh_v.at[pl.ds(0, rem)], bins_t.at[pl.ds(15 * zc, rem)])

        pltpu.sync_copy(simi_hbm, simi_v)
        plsc.subcore_barrier()

        def p1(b, _):
            base = s * edges_per_sc_tile + b * _B
            pltpu.sync_copy(src_hbm.at[pl.ds(base, _B)], sidx)
            pltpu.sync_copy(dst_hbm.at[pl.ds(base, _B)], didx)

            def inner(i, _i):
                iv = sidx[pl.ds(i * 16, 16)]
                ew = plsc.load_gather(simi_v, [iv])
                ebh[pl.ds(i * 16, 16)] = jnp.exp(jnp.maximum(ew, 0.2 * ew))
                ebt[pl.ds(i * 16, 16)] = jnp.exp(jnp.maximum(-ew, -0.2 * ew))
                return _i

            lax.fori_loop(0, _B // 16, inner, None)
            pltpu.sync_copy(ebh, bins_h.at[didx], add=True)
            pltpu.sync_copy(ebt, bins_t.at[didx], add=True)
            return _

        lax.fori_loop(0, nb1, p1, None)
        plsc.subcore_barrier()

        pltpu.sync_copy(bins_h, bh_v)
        pltpu.sync_copy(bins_t, bt_v)

        g = c * 16 + s

        def p3(b, _):
            base = g * edges_per_tile + b * _B
            pltpu.sync_copy(src_hbm.at[pl.ds(base, _B)], sidx)
            pltpu.sync_copy(dst_hbm.at[pl.ds(base, _B)], didx)
            pltpu.async_copy(h_hbm.at[sidx], rows, sem).wait()

            def coef(i, _i):
                iv = sidx[pl.ds(i * 16, 16)]
                dv = didx[pl.ds(i * 16, 16)]
                ew = plsc.load_gather(simi_v, [iv])
                den = jnp.where(ew > 0.0,
                                plsc.load_gather(bh_v, [dv]),
                                plsc.load_gather(bt_v, [dv]))
                ebh[pl.ds(i * 16, 16)] = jnp.exp(jnp.abs(ew)) / den
                return _i

            lax.fori_loop(0, _B // 16, coef, None)

            def scale(i, _i):
                cv = ebh[pl.ds(i * 16, 16)]
                for jj in range(16):
                    cs = cv[jj]
                    r = i * 16 + jj
                    for j in range(d // 16):
                        rows[r, pl.ds(j * 16, 16)] = (
                            rows[r, pl.ds(j * 16, 16)] * cs
                        )
                return _i

            lax.fori_loop(0, _B // 16, scale, None)
            pltpu.sync_copy(rows, acc.at[didx], add=True)
            return _

        lax.fori_loop(0, nb3, p3, None)
        plsc.subcore_barrier()

        @pl.when(s < 15)
        def _():
            pltpu.sync_copy(acc.at[pl.ds(row0, rpt), :],
                            out_hbm.at[pl.ds(c * n + row0, rpt), :])

        @pl.when(s == 15)
        def _():
            pltpu.sync_copy(acc.at[pl.ds(15 * rpt, rpt_last), :],
                            out_hbm.at[pl.ds(c * n + 15 * rpt, rpt_last), :])

    return sc_kernel


def kernel(x, edge_index, simi_weight, W_neigh, b_neigh, W_self, b_self):
    n, d_in = x.shape
    e = edge_index.shape[1]
    d = W_neigh.shape[1]
    src = edge_index[0]
    dst = edge_index[1]

    blk = 1000
    grid = (n // blk,)
    h = pl.pallas_call(
        _tc_linear_body,
        grid=grid,
        in_specs=[
            pl.BlockSpec((blk, d_in), lambda i: (i, 0)),
            pl.BlockSpec((d_in, d), lambda i: (0, 0)),
            pl.BlockSpec((1, d), lambda i: (0, 0)),
        ],
        out_specs=pl.BlockSpec((blk, d), lambda i: (i, 0)),
        out_shape=jax.ShapeDtypeStruct((n, d), jnp.float32),
    )(x, W_neigh, b_neigh.reshape(1, d))

    parts = _make_sc_kernel(n, e, d)(h, src, dst, simi_weight)
    p0 = parts[:n]
    p1 = parts[n:]

    out = pl.pallas_call(
        _tc_combine_body,
        grid=grid,
        in_specs=[
            pl.BlockSpec((blk, d_in), lambda i: (i, 0)),
            pl.BlockSpec((d_in, d), lambda i: (0, 0)),
            pl.BlockSpec((1, d), lambda i: (0, 0)),
            pl.BlockSpec((blk, d), lambda i: (i, 0)),
            pl.BlockSpec((blk, d), lambda i: (i, 0)),
        ],
        out_specs=pl.BlockSpec((blk, d), lambda i: (i, 0)),
        out_shape=jax.ShapeDtypeStruct((n, d), jnp.float32),
    )(x, W_self, b_self.reshape(1, d), p0, p1)
    return out


# speedup vs baseline: 35.8218x; 1.9809x over previous
import functools

import jax
import jax.numpy as jnp
from jax import lax
from jax.experimental import pallas as pl
from jax.experimental.pallas import tpu as pltpu
from jax.experimental.pallas import tpu_sc as plsc

_W = 80
_K1 = 25
_K3 = 8


def _tc_linear_body(x_ref, w_ref, b_ref, o_ref):
    o_ref[...] = (
        jnp.dot(x_ref[...], w_ref[...], preferred_element_type=jnp.float32)
        + b_ref[...]
    )


def _tc_combine_body(x_ref, w_ref, b_ref, p0_ref, p1_ref, o_ref):
    o_ref[...] = (
        jnp.dot(x_ref[...], w_ref[...], preferred_element_type=jnp.float32)
        + b_ref[...]
        + p0_ref[...]
        + p1_ref[...]
    )


def _make_sc_kernel(n, e, d):
    ept1 = e // 16
    ept3 = e // 32
    nch1 = ept1 // (_K1 * _W)
    assert nch1 * _K1 * _W == ept1
    nch3 = ept3 // (_K3 * _W)
    ntail = (ept3 - nch3 * _K3 * _W) // _W
    rpt = (n // 16) // 8 * 8
    rpt_last = n - 15 * rpt
    zc = 640

    mesh = plsc.VectorSubcoreMesh(core_axis_name="c", subcore_axis_name="s")

    @functools.partial(
        pl.kernel,
        out_type=jax.ShapeDtypeStruct((2 * n, d), jnp.float32),
        mesh=mesh,
        compiler_params=pltpu.CompilerParams(
            needs_layout_passes=False, use_tc_tiling_on_sc=False
        ),
        scratch_types=[
            pltpu.VMEM_SHARED((n, d), jnp.float32),
            pltpu.VMEM_SHARED((n,), jnp.float32),
            pltpu.VMEM_SHARED((n,), jnp.float32),
            pltpu.VMEM((n,), jnp.float32),
            pltpu.VMEM((n,), jnp.float32),
            pltpu.VMEM((n,), jnp.float32),
            pltpu.VMEM((zc,), jnp.float32),
            pltpu.VMEM((_K1 * _W,), jnp.int32),
            pltpu.VMEM((_K1 * _W,), jnp.int32),
            pltpu.VMEM((_K1 * _W,), jnp.float32),
            pltpu.VMEM((_K1 * _W,), jnp.float32),
            pltpu.VMEM((_K3 * _W,), jnp.int32),
            pltpu.VMEM((_K3 * _W,), jnp.int32),
            pltpu.VMEM((_K3 * _W,), jnp.float32),
            pltpu.VMEM((_K3 * _W, d), jnp.float32),
            pltpu.VMEM((_W,), jnp.int32),
            pltpu.VMEM((_W,), jnp.int32),
            pltpu.VMEM((_W,), jnp.float32),
            pltpu.VMEM((_W, d), jnp.float32),
            pltpu.SemaphoreType.DMA,
        ],
    )
    def sc_kernel(h_hbm, src_hbm, dst_hbm, simi_hbm, out_hbm,
                  acc, bins_h, bins_t, simi_v, bh_v, bt_v, zb,
                  sidx1, didx1, ebh, ebt,
                  sidx3, didx3, coef3, rows3,
                  sidxt, didxt, coeft, rowst, sem):
        c = lax.axis_index("c")
        s = lax.axis_index("s")
        zeros16 = jnp.zeros((16,), jnp.float32)

        def zero_rowst(i, carry):
            for j in range(d // 16):
                rowst[i, pl.ds(j * 16, 16)] = zeros16
            return carry

        lax.fori_loop(0, _W, zero_rowst, None)

        def zero_zb(i, carry):
            zb[pl.ds(i * 16, 16)] = zeros16
            return carry

        lax.fori_loop(0, zc // 16, zero_zb, None)

        row0 = s * rpt

        @pl.when(s < 15)
        def _():
            for k in range(rpt // _W):
                pltpu.sync_copy(rowst,
                                acc.at[pl.ds(row0 + k * _W, _W), :])
            tail = rpt - (rpt // _W) * _W
            if tail:
                pltpu.sync_copy(
                    rowst.at[pl.ds(0, tail), :],
                    acc.at[pl.ds(row0 + (rpt // _W) * _W, tail), :])
            pltpu.sync_copy(zb, bins_h.at[pl.ds(s * zc, zc)])
            pltpu.sync_copy(zb, bins_t.at[pl.ds(s * zc, zc)])

        @pl.when(s == 15)
        def _():
            for k in range(rpt_last // _W):
                pltpu.sync_copy(rowst,
                                acc.at[pl.ds(15 * rpt + k * _W, _W), :])
            rem = n - 15 * zc
            pltpu.sync_copy(zb.at[pl.ds(0, rem)],
                            bins_h.at[pl.ds(15 * zc, rem)])
            pltpu.sync_copy(zb.at[pl.ds(0, rem)],
                            bins_t.at[pl.ds(15 * zc, rem)])

        pltpu.sync_copy(simi_hbm, simi_v)
        plsc.subcore_barrier()

        def p1(ch, carry):
            base = s * ept1 + ch * (_K1 * _W)
            pltpu.sync_copy(src_hbm.at[pl.ds(base, _K1 * _W)], sidx1)
            pltpu.sync_copy(dst_hbm.at[pl.ds(base, _K1 * _W)], didx1)

            def inner(j, carry2):
                iv = sidx1[pl.ds(j * 16, 16)]
                ew = plsc.load_gather(simi_v, [iv])
                ebh[pl.ds(j * 16, 16)] = jnp.exp(jnp.maximum(ew, 0.2 * ew))
                ebt[pl.ds(j * 16, 16)] = jnp.exp(jnp.maximum(-ew, -0.2 * ew))
                return carry2

            lax.fori_loop(0, _K1 * _W // 16, inner, None)
            pltpu.sync_copy(ebh, bins_h.at[didx1], add=True)
            pltpu.sync_copy(ebt, bins_t.at[didx1], add=True)
            return carry

        lax.fori_loop(0, nch1, p1, None)
        plsc.subcore_barrier()

        pltpu.sync_copy(bins_h, bh_v)
        pltpu.sync_copy(bins_t, bt_v)

        g = c * 16 + s

        def coef_and_scale(sref, dref, cref, rref, j):
            iv = sref[pl.ds(j * 16, 16)]
            dv = dref[pl.ds(j * 16, 16)]
            ew = plsc.load_gather(simi_v, [iv])
            den = jnp.where(ew > 0.0,
                            plsc.load_gather(bh_v, [dv]),
                            plsc.load_gather(bt_v, [dv]))
            cv = jnp.exp(jnp.abs(ew)) / den
            cref[pl.ds(j * 16, 16)] = cv
            for lane in range(16):
                cs = cv[lane]
                for k2 in range(d // 16):
                    rref[j * 16 + lane, pl.ds(k2 * 16, 16)] = (
                        rref[j * 16 + lane, pl.ds(k2 * 16, 16)] * cs
                    )

        def p3(ch, carry):
            base = g * ept3 + ch * (_K3 * _W)
            pltpu.sync_copy(src_hbm.at[pl.ds(base, _K3 * _W)], sidx3)
            pltpu.sync_copy(dst_hbm.at[pl.ds(base, _K3 * _W)], didx3)
            pltpu.async_copy(h_hbm.at[sidx3], rows3, sem).wait()

            def body(j, carry2):
                coef_and_scale(sidx3, didx3, coef3, rows3, j)
                return carry2

            lax.fori_loop(0, _K3 * _W // 16, body, None)
            pltpu.sync_copy(rows3, acc.at[didx3], add=True)
            return carry

        lax.fori_loop(0, nch3, p3, None)

        def p3t(t, carry):
            base = g * ept3 + nch3 * (_K3 * _W) + t * _W
            pltpu.sync_copy(src_hbm.at[pl.ds(base, _W)], sidxt)
            pltpu.sync_copy(dst_hbm.at[pl.ds(base, _W)], didxt)
            pltpu.async_copy(h_hbm.at[sidxt], rowst, sem).wait()

            def bodyt(j, carry2):
                coef_and_scale(sidxt, didxt, coeft, rowst, j)
                return carry2

            lax.fori_loop(0, _W // 16, bodyt, None)
            pltpu.sync_copy(rowst, acc.at[didxt], add=True)
            return carry

        lax.fori_loop(0, ntail, p3t, None)
        plsc.subcore_barrier()

        @pl.when(s < 15)
        def _():
            pltpu.sync_copy(acc.at[pl.ds(row0, rpt), :],
                            out_hbm.at[pl.ds(c * n + row0, rpt), :])

        @pl.when(s == 15)
        def _():
            pltpu.sync_copy(acc.at[pl.ds(15 * rpt, rpt_last), :],
                            out_hbm.at[pl.ds(c * n + 15 * rpt, rpt_last), :])

    return sc_kernel


def kernel(x, edge_index, simi_weight, W_neigh, b_neigh, W_self, b_self):
    n, d_in = x.shape
    e = edge_index.shape[1]
    d = W_neigh.shape[1]
    src = edge_index[0]
    dst = edge_index[1]

    blk = 1000
    grid = (n // blk,)
    h = pl.pallas_call(
        _tc_linear_body,
        grid=grid,
        in_specs=[
            pl.BlockSpec((blk, d_in), lambda i: (i, 0)),
            pl.BlockSpec((d_in, d), lambda i: (0, 0)),
            pl.BlockSpec((1, d), lambda i: (0, 0)),
        ],
        out_specs=pl.BlockSpec((blk, d), lambda i: (i, 0)),
        out_shape=jax.ShapeDtypeStruct((n, d), jnp.float32),
    )(x, W_neigh, b_neigh.reshape(1, d))

    parts = _make_sc_kernel(n, e, d)(h, src, dst, simi_weight)
    p0 = parts[:n]
    p1 = parts[n:]

    out = pl.pallas_call(
        _tc_combine_body,
        grid=grid,
        in_specs=[
            pl.BlockSpec((blk, d_in), lambda i: (i, 0)),
            pl.BlockSpec((d_in, d), lambda i: (0, 0)),
            pl.BlockSpec((1, d), lambda i: (0, 0)),
            pl.BlockSpec((blk, d), lambda i: (i, 0)),
            pl.BlockSpec((blk, d), lambda i: (i, 0)),
        ],
        out_specs=pl.BlockSpec((blk, d), lambda i: (i, 0)),
        out_shape=jax.ShapeDtypeStruct((n, d), jnp.float32),
    )(x, W_self, b_self.reshape(1, d), p0, p1)
    return out
